# Initial kernel scaffold; baseline (speedup 1.0000x reference)
import jax
import jax.numpy as jnp
from jax.experimental import pallas as pl


def kernel(xyz, features):
    raise NotImplementedError("write your pallas kernel here")


# speedup vs baseline: 21.4825x; 21.4825x over previous
import functools

import jax
import jax.numpy as jnp
from jax import lax
from jax.experimental import pallas as pl
from jax.experimental.pallas import tpu as pltpu
from jax.experimental.pallas import tpu_sc as plsc

_P = 512
_LANES = 128


def _fps_body(x_ref, y_ref, z_ref, d0_ref, f0_ref,
              inds_ref, sx_ref, sy_ref, sz_ref, d_ref):
    B, Np = x_ref.shape
    d_ref[...] = d0_ref[...]
    lane = lax.broadcasted_iota(jnp.int32, (B, Np), 1)
    lane_p = lax.broadcasted_iota(jnp.int32, (B, _P), 1)

    def it(i, carry):
        far, inds, sx, sy, sz = carry
        xv = x_ref[...]
        yv = y_ref[...]
        zv = z_ref[...]
        eq = lane == far
        zero = jnp.zeros((B, Np), jnp.float32)
        cx = jnp.sum(jnp.where(eq, xv, zero), axis=1, keepdims=True)
        cy = jnp.sum(jnp.where(eq, yv, zero), axis=1, keepdims=True)
        cz = jnp.sum(jnp.where(eq, zv, zero), axis=1, keepdims=True)
        dx = xv - cx
        dy = yv - cy
        dz = zv - cz
        dist = dx * dx + dy * dy + dz * dz
        d = jnp.minimum(d_ref[...], dist)
        d_ref[...] = d
        m = jnp.max(d, axis=1, keepdims=True)
        nf = jnp.min(jnp.where(d == m, lane, Np), axis=1, keepdims=True)
        sel = lane_p == i
        inds = jnp.where(sel, far, inds)
        sx = jnp.where(sel, cx, sx)
        sy = jnp.where(sel, cy, sy)
        sz = jnp.where(sel, cz, sz)
        return (nf, inds, sx, sy, sz)

    init = (f0_ref[...],
            jnp.zeros((B, _P), jnp.int32),
            jnp.zeros((B, _P), jnp.float32),
            jnp.zeros((B, _P), jnp.float32),
            jnp.zeros((B, _P), jnp.float32))
    _, inds, sx, sy, sz = lax.fori_loop(0, _P, it, init)
    inds_ref[...] = inds
    sx_ref[...] = sx
    sy_ref[...] = sy
    sz_ref[...] = sz


def _fps_tc(x, y, z, d0, f0):
    B, Np = x.shape
    return pl.pallas_call(
        _fps_body,
        out_shape=[
            jax.ShapeDtypeStruct((B, _P), jnp.int32),
            jax.ShapeDtypeStruct((B, _P), jnp.float32),
            jax.ShapeDtypeStruct((B, _P), jnp.float32),
            jax.ShapeDtypeStruct((B, _P), jnp.float32),
        ],
        scratch_shapes=[pltpu.VMEM((B, Np), jnp.float32)],
    )(x, y, z, d0, f0)


def _gather_sc(feat2d, inds):
    R, N = feat2d.shape
    B = inds.shape[0]
    NW = 32
    rpw = R // NW
    mesh = plsc.VectorSubcoreMesh(core_axis_name="c", subcore_axis_name="s")

    @functools.partial(
        pl.kernel,
        out_type=jax.ShapeDtypeStruct((R, _P), jnp.float32),
        mesh=mesh,
        compiler_params=pltpu.CompilerParams(needs_layout_passes=False),
        scratch_types=[
            pltpu.VMEM((_P,), jnp.int32),
            pltpu.VMEM((N,), jnp.float32),
            pltpu.VMEM((_P,), jnp.float32),
        ],
    )
    def gather_k(feat_hbm, idx_hbm, out_hbm, idx_v, row_v, out_v):
        cid = lax.axis_index("c")
        sid = lax.axis_index("s")
        wid = sid * 2 + cid
        row0 = wid * rpw
        b = row0 // (R // B)
        pltpu.sync_copy(idx_hbm.at[b], idx_v)

        def row_body(r, _):
            row_id = row0 + r
            pltpu.sync_copy(feat_hbm.at[row_id], row_v)
            for k in range(_P // 16):
                iv = idx_v[pl.ds(k * 16, 16)]
                out_v[pl.ds(k * 16, 16)] = plsc.load_gather(row_v, [iv])
            pltpu.sync_copy(out_v, out_hbm.at[row_id])
            return 0

        lax.fori_loop(0, rpw, row_body, 0)

    return gather_k(feat2d, inds)


def kernel(xyz, features):
    B, N, _ = xyz.shape
    C = features.shape[1]
    Np = ((N + _LANES - 1) // _LANES) * _LANES

    pad = Np - N
    x = jnp.pad(xyz[:, :, 0], ((0, 0), (0, pad)))
    y = jnp.pad(xyz[:, :, 1], ((0, 0), (0, pad)))
    z = jnp.pad(xyz[:, :, 2], ((0, 0), (0, pad)))
    d0 = jnp.concatenate(
        [jnp.full((B, N), 1e10, jnp.float32),
         jnp.full((B, pad), -jnp.inf, jnp.float32)], axis=1)
    f0 = jax.random.randint(jax.random.key(1), (B,), 0, N,
                            dtype=jnp.int32)[:, None]

    inds, sx, sy, sz = _fps_tc(x, y, z, d0, f0)
    new_xyz = jnp.stack([sx, sy, sz], axis=-1)

    feat2d = features.reshape(B * C, N)
    new_features = _gather_sc(feat2d, inds).reshape(B, C, _P)
    return (new_xyz, new_features, inds)


# speedup vs baseline: 27.6773x; 1.2884x over previous
import functools

import jax
import jax.numpy as jnp
from jax import lax
from jax.experimental import pallas as pl
from jax.experimental.pallas import tpu as pltpu
from jax.experimental.pallas import tpu_sc as plsc

_P = 512
_LANES = 128


def _fps_body(x_ref, y_ref, z_ref, d0_ref, f0_ref,
              inds_ref, sx_ref, sy_ref, sz_ref, d_ref):
    B, Np = x_ref.shape
    T = Np // _LANES
    d_ref[...] = d0_ref[...]
    lane8 = lax.broadcasted_iota(jnp.int32, (B, _LANES), 1)
    lane_p = lax.broadcasted_iota(jnp.int32, (B, _P), 1)
    lane_full = lax.broadcasted_iota(jnp.int32, (B, Np), 1)

    far0 = f0_ref[...]
    eq = lane_full == far0
    zero_full = jnp.zeros((B, Np), jnp.float32)
    cx0 = jnp.sum(jnp.where(eq, x_ref[...], zero_full), axis=1, keepdims=True)
    cy0 = jnp.sum(jnp.where(eq, y_ref[...], zero_full), axis=1, keepdims=True)
    cz0 = jnp.sum(jnp.where(eq, z_ref[...], zero_full), axis=1, keepdims=True)

    zero128 = jnp.zeros((B, _LANES), jnp.float32)

    def it(i, carry):
        far, cx, cy, cz, inds, sx, sy, sz = carry
        sel = lane_p == i
        inds = jnp.where(sel, far, inds)
        sx = jnp.where(sel, cx, sx)
        sy = jnp.where(sel, cy, sy)
        sz = jnp.where(sel, cz, sz)

        run_v = jnp.full((B, _LANES), -jnp.inf, jnp.float32)
        run_t = jnp.zeros((B, _LANES), jnp.int32)
        run_x = zero128
        run_y = zero128
        run_z = zero128
        for t in range(T):
            s = pl.ds(t * _LANES, _LANES)
            xv = x_ref[:, s]
            yv = y_ref[:, s]
            zv = z_ref[:, s]
            dx = xv - cx
            dy = yv - cy
            dz = zv - cz
            dist = dx * dx + dy * dy + dz * dz
            nd = jnp.minimum(d_ref[:, s], dist)
            d_ref[:, s] = nd
            cond = nd > run_v
            run_v = jnp.where(cond, nd, run_v)
            run_t = jnp.where(cond, t, run_t)
            run_x = jnp.where(cond, xv, run_x)
            run_y = jnp.where(cond, yv, run_y)
            run_z = jnp.where(cond, zv, run_z)

        gidx = run_t * _LANES + lane8
        m = jnp.max(run_v, axis=1, keepdims=True)
        ksel = jnp.where(run_v == m, gidx, Np)
        nf = jnp.min(ksel, axis=1, keepdims=True)
        fm = ksel == nf
        ncx = jnp.sum(jnp.where(fm, run_x, zero128), axis=1, keepdims=True)
        ncy = jnp.sum(jnp.where(fm, run_y, zero128), axis=1, keepdims=True)
        ncz = jnp.sum(jnp.where(fm, run_z, zero128), axis=1, keepdims=True)
        return (nf, ncx, ncy, ncz, inds, sx, sy, sz)

    init = (far0, cx0, cy0, cz0,
            jnp.zeros((B, _P), jnp.int32),
            jnp.zeros((B, _P), jnp.float32),
            jnp.zeros((B, _P), jnp.float32),
            jnp.zeros((B, _P), jnp.float32))
    _, _, _, _, inds, sx, sy, sz = lax.fori_loop(0, _P, it, init)
    inds_ref[...] = inds
    sx_ref[...] = sx
    sy_ref[...] = sy
    sz_ref[...] = sz


def _fps_tc(x, y, z, d0, f0):
    B, Np = x.shape
    return pl.pallas_call(
        _fps_body,
        out_shape=[
            jax.ShapeDtypeStruct((B, _P), jnp.int32),
            jax.ShapeDtypeStruct((B, _P), jnp.float32),
            jax.ShapeDtypeStruct((B, _P), jnp.float32),
            jax.ShapeDtypeStruct((B, _P), jnp.float32),
        ],
        scratch_shapes=[pltpu.VMEM((B, Np), jnp.float32)],
    )(x, y, z, d0, f0)


def _gather_sc(features, inds):
    B, C, N = features.shape
    R = B * C
    NW = 32
    rpw = R // NW
    mesh = plsc.VectorSubcoreMesh(core_axis_name="c", subcore_axis_name="s")

    @functools.partial(
        pl.kernel,
        out_type=jax.ShapeDtypeStruct((B, C, _P), jnp.float32),
        mesh=mesh,
        compiler_params=pltpu.CompilerParams(needs_layout_passes=False),
        scratch_types=[
            pltpu.VMEM((_P,), jnp.int32),
            pltpu.VMEM((N,), jnp.float32),
            pltpu.VMEM((_P,), jnp.float32),
        ],
    )
    def gather_k(feat_hbm, idx_hbm, out_hbm, idx_v, row_v, out_v):
        cid = lax.axis_index("c")
        sid = lax.axis_index("s")
        wid = sid * 2 + cid
        row0 = wid * rpw
        b = row0 // C
        c0 = row0 % C
        pltpu.sync_copy(idx_hbm.at[b], idx_v)

        def row_body(r, _):
            c = c0 + r
            pltpu.sync_copy(feat_hbm.at[b, c], row_v)
            for k in range(_P // 16):
                iv = idx_v[pl.ds(k * 16, 16)]
                out_v[pl.ds(k * 16, 16)] = plsc.load_gather(row_v, [iv])
            pltpu.sync_copy(out_v, out_hbm.at[b, c])
            return 0

        lax.fori_loop(0, rpw, row_body, 0)

    return gather_k(features, inds)


def kernel(xyz, features):
    B, N, _ = xyz.shape
    Np = ((N + _LANES - 1) // _LANES) * _LANES

    pad = Np - N
    x = jnp.pad(xyz[:, :, 0], ((0, 0), (0, pad)))
    y = jnp.pad(xyz[:, :, 1], ((0, 0), (0, pad)))
    z = jnp.pad(xyz[:, :, 2], ((0, 0), (0, pad)))
    d0 = jnp.concatenate(
        [jnp.full((B, N), 1e10, jnp.float32),
         jnp.full((B, pad), -jnp.inf, jnp.float32)], axis=1)
    f0 = jax.random.randint(jax.random.key(1), (B,), 0, N,
                            dtype=jnp.int32)[:, None]

    inds, sx, sy, sz = _fps_tc(x, y, z, d0, f0)
    new_xyz = jnp.stack([sx, sy, sz], axis=-1)

    new_features = _gather_sc(features, inds)
    return (new_xyz, new_features, inds)
